# Initial kernel scaffold; baseline (speedup 1.0000x reference)
import jax
import jax.numpy as jnp
from jax.experimental import pallas as pl


def kernel(problems, initial, W1, b1, Wv, bv, logZ):
    raise NotImplementedError("write your pallas kernel here")


# speedup vs baseline: 4.3423x; 4.3423x over previous
import functools

import jax
import jax.numpy as jnp
from jax import lax
from jax.experimental import pallas as pl
from jax.experimental.pallas import tpu as pltpu
from jax.experimental.pallas import tpu_sc as plsc

K_BT = 3
M_REC = 3
ROWS_A = 8
MXU_M = 5000


def _mxu_body(x_ref, w1_ref, b1_ref, wv_ref, o_ref):
    x = x_ref[...]
    h = jax.lax.dot_general(x, w1_ref[...], (((1,), (0,)), ((), ())),
                            preferred_element_type=jnp.float32)
    h = jnp.maximum(h + b1_ref[...], 0.0)
    o_ref[...] = jax.lax.dot_general(h, wv_ref[...], (((1,), (0,)), ((), ())),
                                     preferred_element_type=jnp.float32)


def _stats_body(n_cities, s_ref, p_ref, stats_ref, idx_ref):
    scores = s_ref[...]
    m = jnp.max(scores, axis=1, keepdims=True)
    se = jnp.sum(jnp.exp(scores - m), axis=1, keepdims=True)
    lse = m + jnp.log(se)
    rowsum = jnp.sum(scores, axis=1, keepdims=True)

    iota = lax.broadcasted_iota(jnp.int32, scores.shape, 1)
    s = scores
    tv, ti = [], []
    for _ in range(K_BT):
        mv = jnp.max(s, axis=1, keepdims=True)
        mi = jnp.min(jnp.where(s == mv, iota, n_cities), axis=1,
                     keepdims=True)
        tv.append(mv)
        ti.append(mi)
        s = jnp.where(iota == mi, -jnp.inf, s)

    col = lax.broadcasted_iota(jnp.int32, stats_ref.shape, 1)
    stats = jnp.zeros(stats_ref.shape, jnp.float32)
    for c, val in ((0, lse), (1, rowsum), (2, tv[0]), (3, tv[1]), (4, tv[2]),
                   (5, jnp.full_like(lse, p_ref[1])),
                   (6, jnp.full_like(lse, p_ref[0]))):
        stats = jnp.where(col == c, val, stats)
    stats_ref[...] = stats
    idx = jnp.zeros(idx_ref.shape, jnp.int32)
    for c, val in ((0, ti[0]), (1, ti[1]), (2, ti[2])):
        idx = jnp.where(col == c, val, idx)
    idx_ref[...] = idx


def _sqrt16(x):
    i = plsc.bitcast(x, jnp.int32)
    y = plsc.bitcast(jnp.int32(0x5F3759DF) - (i >> 1), jnp.float32)
    for _ in range(3):
        y = y * (1.5 - 0.5 * x * y * y)
    return x * y


def _sc_body(n_cities, px_hbm, py_hbm, ini_hbm, sc_hbm, st_hbm, bt_hbm,
             out_hbm, ini_v, px_v, py_v, tcx_v, tcy_v, sc_v, st_v, bt_v,
             out_v):
    n = n_cities
    wid = lax.axis_index("c") * 16 + lax.axis_index("s")
    iota16 = lax.iota(jnp.int32, 16)
    zeros16 = jnp.zeros((16,), jnp.float32)

    for r in range(2):
        b = wid + 32 * r
        pltpu.sync_copy(ini_hbm.at[b], ini_v)
        pltpu.sync_copy(px_hbm.at[b], px_v)
        pltpu.sync_copy(py_hbm.at[b], py_v)
        pltpu.sync_copy(sc_hbm.at[b], sc_v)
        pltpu.sync_copy(st_hbm.at[b], st_v)
        pltpu.sync_copy(bt_hbm.at[b], bt_v)

        def chunk(k, acc):
            off = k * 16
            ia = ini_v[pl.ds(off, 16)]
            ip1 = iota16 + (off + 1)
            ip1 = jnp.where(ip1 == n, 0, ip1)
            ib = plsc.load_gather(ini_v, [ip1])
            xa = plsc.load_gather(px_v, [ia])
            ya = plsc.load_gather(py_v, [ia])
            xb = plsc.load_gather(px_v, [ib])
            yb = plsc.load_gather(py_v, [ib])
            tcx_v[pl.ds(off, 16)] = xa
            tcy_v[pl.ds(off, 16)] = ya
            dx = xa - xb
            dy = ya - yb
            return acc + _sqrt16(dx * dx + dy * dy + 1e-10)

        rem16 = lax.fori_loop(0, n // 16, chunk, zeros16, unroll=2)
        remb = jnp.sum(rem16)

        st16 = st_v[pl.ds(0, 16)]
        bt16 = bt_v[pl.ds(0, 16)]
        lse = st16[0]
        logz = st16[5]
        outvec = zeros16
        for t in range(K_BT):
            lo = bt16[t]
            log_pb = st16[2 + t] - lse
            lom1 = jnp.where(lo == 0, n - 1, lo - 1)
            for j in range(M_REC):
                seg = 100 * (j + 1)
                hi = lo + seg
                w_end = jnp.minimum(hi, n)
                kmax = w_end - lo
                wmod = jnp.where(w_end == n, 0, w_end)

                def cchunk(c, carry, hi=hi, kmax=kmax, wmod=wmod, lo=lo,
                           lom1=lom1):
                    dacc, wacc = carry
                    kv = iota16 + c * 16
                    m_edge = kv <= kmax
                    m_win = kv < kmax
                    rev = jnp.clip(hi - 1 - kv, 0, n - 1)
                    tca = jnp.where(kv == 0, lom1, jnp.clip(hi - kv, 0, n - 1))
                    tcb = jnp.where(kv == kmax, wmod, rev)
                    olda = jnp.where(kv == 0, lom1,
                                     jnp.clip(lo - 1 + kv, 0, n - 1))
                    oldb = jnp.where(kv == kmax, wmod,
                                     jnp.clip(lo + kv, 0, n - 1))
                    nax = plsc.load_gather(tcx_v, [tca])
                    nay = plsc.load_gather(tcy_v, [tca])
                    nbx = plsc.load_gather(tcx_v, [tcb])
                    nby = plsc.load_gather(tcy_v, [tcb])
                    oax = plsc.load_gather(tcx_v, [olda])
                    oay = plsc.load_gather(tcy_v, [olda])
                    obx = plsc.load_gather(tcx_v, [oldb])
                    oby = plsc.load_gather(tcy_v, [oldb])
                    ndx = nax - nbx
                    ndy = nay - nby
                    odx = oax - obx
                    ody = oay - oby
                    nd = _sqrt16(ndx * ndx + ndy * ndy + 1e-10)
                    od = _sqrt16(odx * odx + ody * ody + 1e-10)
                    sv = plsc.load_gather(sc_v, [rev])
                    dacc = dacc + jnp.where(m_edge, nd - od, 0.0)
                    wacc = wacc + jnp.where(m_win, sv, 0.0)
                    return dacc, wacc

                d16, w16 = lax.fori_loop(0, 19, cchunk, (zeros16, zeros16))
                delta = jnp.sum(d16)
                win_logp = jnp.sum(w16) - kmax.astype(jnp.float32) * lse
                tb = logz + win_logp + remb + delta - log_pb
                outvec = jnp.where(iota16 == t * M_REC + j, tb * tb, outvec)
        outvec = jnp.where(iota16 == 9, remb, outvec)
        out_v[pl.ds(0, 16)] = outvec
        pltpu.sync_copy(out_v, out_hbm.at[b])


def _combine_body(batch, n_cities, tb_ref, st_ref, out_ref):
    tb = tb_ref[...]
    st = st_ref[...]
    col = lax.broadcasted_iota(jnp.int32, tb.shape, 1)
    loss_tb = jnp.sum(jnp.where(col < K_BT * M_REC, tb, 0.0)) / (
        batch * K_BT * M_REC)
    pred = st[:, 1:2] / n_cities + st[:, 6:7]
    rem = tb[:, 9:10]
    v_loss = jnp.sum((pred - rem) * (pred - rem)) / batch
    out_ref[...] = jnp.full(out_ref.shape, loss_tb + 0.1 * v_loss,
                            jnp.float32)


def kernel(problems, initial, W1, b1, Wv, bv, logZ):
    batch, n, _ = problems.shape
    f32 = jnp.float32
    pf = problems.reshape(batch * n, 2)
    px = problems[:, :, 0]
    py = problems[:, :, 1]

    sflat = pl.pallas_call(
        _mxu_body,
        grid=(batch * n // MXU_M,),
        in_specs=[
            pl.BlockSpec((MXU_M, 2), lambda i: (i, 0)),
            pl.BlockSpec((2, 32), lambda i: (0, 0)),
            pl.BlockSpec((1, 32), lambda i: (0, 0)),
            pl.BlockSpec((32, 1), lambda i: (0, 0)),
        ],
        out_specs=pl.BlockSpec((MXU_M, 1), lambda i: (i, 0)),
        out_shape=jax.ShapeDtypeStruct((batch * n, 1), f32),
    )(pf, W1, b1.reshape(1, 32), Wv)
    scores = sflat.reshape(batch, n)

    pack = jnp.concatenate([bv, logZ, jnp.zeros((6,), f32)])
    stats, bt = pl.pallas_call(
        functools.partial(_stats_body, n),
        grid=(batch // ROWS_A,),
        in_specs=[
            pl.BlockSpec((ROWS_A, n), lambda i: (i, 0)),
            pl.BlockSpec(memory_space=pltpu.SMEM),
        ],
        out_specs=[
            pl.BlockSpec((ROWS_A, 128), lambda i: (i, 0)),
            pl.BlockSpec((ROWS_A, 128), lambda i: (i, 0)),
        ],
        out_shape=[
            jax.ShapeDtypeStruct((batch, 128), f32),
            jax.ShapeDtypeStruct((batch, 128), jnp.int32),
        ],
    )(scores, pack)

    mesh = plsc.VectorSubcoreMesh(core_axis_name="c", subcore_axis_name="s")
    tbrem = pl.kernel(
        functools.partial(_sc_body, n),
        out_type=jax.ShapeDtypeStruct((batch, 16), f32),
        mesh=mesh,
        compiler_params=pltpu.CompilerParams(needs_layout_passes=False),
        scratch_types=[
            pltpu.VMEM((n,), jnp.int32),
            pltpu.VMEM((n,), f32),
            pltpu.VMEM((n,), f32),
            pltpu.VMEM((n,), f32),
            pltpu.VMEM((n,), f32),
            pltpu.VMEM((n,), f32),
            pltpu.VMEM((128,), f32),
            pltpu.VMEM((128,), jnp.int32),
            pltpu.VMEM((16,), f32),
        ],
    )(px, py, initial, scores, stats, bt)

    out = pl.pallas_call(
        functools.partial(_combine_body, batch, n),
        in_specs=[
            pl.BlockSpec((batch, 16), lambda: (0, 0)),
            pl.BlockSpec((batch, 128), lambda: (0, 0)),
        ],
        out_specs=pl.BlockSpec((8, 128), lambda: (0, 0)),
        out_shape=jax.ShapeDtypeStruct((8, 128), f32),
    )(tbrem, stats)
    return out[0, 0]


# speedup vs baseline: 26.6711x; 6.1421x over previous
import functools

import jax
import jax.numpy as jnp
from jax import lax
from jax.experimental import pallas as pl
from jax.experimental.pallas import tpu as pltpu
from jax.experimental.pallas import tpu_sc as plsc

K_BT = 3
M_REC = 3
ROWS_A = 8
MXU_M = 5000


def _mxu_body(pt_ref, w1t_ref, b1t_ref, wvt_ref, o_ref):
    w1t = w1t_ref[...]
    b1t = b1t_ref[...]
    wvt = wvt_ref[...]
    for r in range(ROWS_A):
        xt = pt_ref[r]
        ht = jax.lax.dot_general(w1t, xt, (((1,), (0,)), ((), ())),
                                 preferred_element_type=jnp.float32)
        ht = jnp.maximum(ht + b1t, 0.0)
        st = jax.lax.dot_general(wvt, ht, (((1,), (0,)), ((), ())),
                                 preferred_element_type=jnp.float32)
        o_ref[r:r + 1, :] = st


def _stats_body(n_cities, s_ref, p_ref, stats_ref, idx_ref):
    scores = s_ref[...]
    m = jnp.max(scores, axis=1, keepdims=True)
    se = jnp.sum(jnp.exp(scores - m), axis=1, keepdims=True)
    lse = m + jnp.log(se)
    rowsum = jnp.sum(scores, axis=1, keepdims=True)

    iota = lax.broadcasted_iota(jnp.int32, scores.shape, 1)
    s = scores
    tv, ti = [], []
    for _ in range(K_BT):
        mv = jnp.max(s, axis=1, keepdims=True)
        mi = jnp.min(jnp.where(s == mv, iota, n_cities), axis=1,
                     keepdims=True)
        tv.append(mv)
        ti.append(mi)
        s = jnp.where(iota == mi, -jnp.inf, s)

    col = lax.broadcasted_iota(jnp.int32, stats_ref.shape, 1)
    stats = jnp.zeros(stats_ref.shape, jnp.float32)
    for c, val in ((0, lse), (1, rowsum), (2, tv[0]), (3, tv[1]), (4, tv[2]),
                   (5, jnp.full_like(lse, p_ref[1])),
                   (6, jnp.full_like(lse, p_ref[0]))):
        stats = jnp.where(col == c, val, stats)
    stats_ref[...] = stats
    idx = jnp.zeros(idx_ref.shape, jnp.int32)
    for c, val in ((0, ti[0]), (1, ti[1]), (2, ti[2])):
        idx = jnp.where(col == c, val, idx)
    idx_ref[...] = idx


def _sqrt16(x):
    i = plsc.bitcast(x, jnp.int32)
    y = plsc.bitcast(jnp.int32(0x5F3759DF) - (i >> 1), jnp.float32)
    for _ in range(3):
        y = y * (1.5 - 0.5 * x * y * y)
    return x * y


def _sc_body(n_cities, px_hbm, py_hbm, ini_hbm, sc_hbm, st_hbm, bt_hbm,
             out_hbm, ini_v, px_v, py_v, tcx_v, tcy_v, sc_v, st_v, bt_v,
             out_v):
    n = n_cities
    wid = lax.axis_index("c") * 16 + lax.axis_index("s")
    iota16 = lax.iota(jnp.int32, 16)
    zeros16 = jnp.zeros((16,), jnp.float32)

    for r in range(2):
        b = wid + 32 * r
        pltpu.sync_copy(ini_hbm.at[b], ini_v)
        pltpu.sync_copy(px_hbm.at[b], px_v)
        pltpu.sync_copy(py_hbm.at[b], py_v)
        pltpu.sync_copy(sc_hbm.at[b], sc_v)
        pltpu.sync_copy(st_hbm.at[b], st_v)
        pltpu.sync_copy(bt_hbm.at[b], bt_v)

        def chunk(k, acc):
            off = k * 16
            ia = ini_v[pl.ds(off, 16)]
            ip1 = iota16 + (off + 1)
            ip1 = jnp.where(ip1 == n, 0, ip1)
            ib = plsc.load_gather(ini_v, [ip1])
            xa = plsc.load_gather(px_v, [ia])
            ya = plsc.load_gather(py_v, [ia])
            xb = plsc.load_gather(px_v, [ib])
            yb = plsc.load_gather(py_v, [ib])
            tcx_v[pl.ds(off, 16)] = xa
            tcy_v[pl.ds(off, 16)] = ya
            dx = xa - xb
            dy = ya - yb
            return acc + _sqrt16(dx * dx + dy * dy + 1e-10)

        rem16 = lax.fori_loop(0, n // 16, chunk, zeros16, unroll=2)
        remb = jnp.sum(rem16)

        st16 = st_v[pl.ds(0, 16)]
        bt16 = bt_v[pl.ds(0, 16)]
        lse = st16[0]
        logz = st16[5]
        outvec = zeros16
        for t in range(K_BT):
            lo = bt16[t]
            log_pb = st16[2 + t] - lse
            lom1 = jnp.where(lo == 0, n - 1, lo - 1)
            for j in range(M_REC):
                seg = 100 * (j + 1)
                hi = lo + seg
                w_end = jnp.minimum(hi, n)
                kmax = w_end - lo
                wmod = jnp.where(w_end == n, 0, w_end)

                def cchunk(c, carry, hi=hi, kmax=kmax, wmod=wmod, lo=lo,
                           lom1=lom1):
                    dacc, wacc = carry
                    kv = iota16 + c * 16
                    m_edge = kv <= kmax
                    m_win = kv < kmax
                    rev = jnp.clip(hi - 1 - kv, 0, n - 1)
                    tca = jnp.where(kv == 0, lom1, jnp.clip(hi - kv, 0, n - 1))
                    tcb = jnp.where(kv == kmax, wmod, rev)
                    olda = jnp.where(kv == 0, lom1,
                                     jnp.clip(lo - 1 + kv, 0, n - 1))
                    oldb = jnp.where(kv == kmax, wmod,
                                     jnp.clip(lo + kv, 0, n - 1))
                    nax = plsc.load_gather(tcx_v, [tca])
                    nay = plsc.load_gather(tcy_v, [tca])
                    nbx = plsc.load_gather(tcx_v, [tcb])
                    nby = plsc.load_gather(tcy_v, [tcb])
                    oax = plsc.load_gather(tcx_v, [olda])
                    oay = plsc.load_gather(tcy_v, [olda])
                    obx = plsc.load_gather(tcx_v, [oldb])
                    oby = plsc.load_gather(tcy_v, [oldb])
                    ndx = nax - nbx
                    ndy = nay - nby
                    odx = oax - obx
                    ody = oay - oby
                    nd = _sqrt16(ndx * ndx + ndy * ndy + 1e-10)
                    od = _sqrt16(odx * odx + ody * ody + 1e-10)
                    sv = plsc.load_gather(sc_v, [rev])
                    dacc = dacc + jnp.where(m_edge, nd - od, 0.0)
                    wacc = wacc + jnp.where(m_win, sv, 0.0)
                    return dacc, wacc

                d16, w16 = lax.fori_loop(0, 19, cchunk, (zeros16, zeros16))
                delta = jnp.sum(d16)
                win_logp = jnp.sum(w16) - kmax.astype(jnp.float32) * lse
                tb = logz + win_logp + remb + delta - log_pb
                outvec = jnp.where(iota16 == t * M_REC + j, tb * tb, outvec)
        outvec = jnp.where(iota16 == 9, remb, outvec)
        out_v[pl.ds(0, 16)] = outvec
        pltpu.sync_copy(out_v, out_hbm.at[b])


def _combine_body(batch, n_cities, tb_ref, st_ref, out_ref):
    tb = tb_ref[...]
    st = st_ref[...]
    col = lax.broadcasted_iota(jnp.int32, tb.shape, 1)
    loss_tb = jnp.sum(jnp.where(col < K_BT * M_REC, tb, 0.0)) / (
        batch * K_BT * M_REC)
    pred = st[:, 1:2] / n_cities + st[:, 6:7]
    rem = tb[:, 9:10]
    v_loss = jnp.sum((pred - rem) * (pred - rem)) / batch
    out_ref[...] = jnp.full(out_ref.shape, loss_tb + 0.1 * v_loss,
                            jnp.float32)


def kernel(problems, initial, W1, b1, Wv, bv, logZ):
    batch, n, _ = problems.shape
    f32 = jnp.float32
    pt = problems.transpose(0, 2, 1)
    px = pt[:, 0, :]
    py = pt[:, 1, :]

    scores = pl.pallas_call(
        _mxu_body,
        grid=(batch // ROWS_A,),
        in_specs=[
            pl.BlockSpec((ROWS_A, 2, n), lambda i: (i, 0, 0)),
            pl.BlockSpec((32, 2), lambda i: (0, 0)),
            pl.BlockSpec((32, 1), lambda i: (0, 0)),
            pl.BlockSpec((1, 32), lambda i: (0, 0)),
        ],
        out_specs=pl.BlockSpec((ROWS_A, n), lambda i: (i, 0)),
        out_shape=jax.ShapeDtypeStruct((batch, n), f32),
    )(pt, W1.T, b1.reshape(32, 1), Wv.T)

    pack = jnp.concatenate([bv, logZ, jnp.zeros((6,), f32)])
    stats, bt = pl.pallas_call(
        functools.partial(_stats_body, n),
        grid=(batch // ROWS_A,),
        in_specs=[
            pl.BlockSpec((ROWS_A, n), lambda i: (i, 0)),
            pl.BlockSpec(memory_space=pltpu.SMEM),
        ],
        out_specs=[
            pl.BlockSpec((ROWS_A, 128), lambda i: (i, 0)),
            pl.BlockSpec((ROWS_A, 128), lambda i: (i, 0)),
        ],
        out_shape=[
            jax.ShapeDtypeStruct((batch, 128), f32),
            jax.ShapeDtypeStruct((batch, 128), jnp.int32),
        ],
    )(scores, pack)

    mesh = plsc.VectorSubcoreMesh(core_axis_name="c", subcore_axis_name="s")
    tbrem = pl.kernel(
        functools.partial(_sc_body, n),
        out_type=jax.ShapeDtypeStruct((batch, 16), f32),
        mesh=mesh,
        compiler_params=pltpu.CompilerParams(needs_layout_passes=False),
        scratch_types=[
            pltpu.VMEM((n,), jnp.int32),
            pltpu.VMEM((n,), f32),
            pltpu.VMEM((n,), f32),
            pltpu.VMEM((n,), f32),
            pltpu.VMEM((n,), f32),
            pltpu.VMEM((n,), f32),
            pltpu.VMEM((128,), f32),
            pltpu.VMEM((128,), jnp.int32),
            pltpu.VMEM((16,), f32),
        ],
    )(px, py, initial, scores, stats, bt)

    out = pl.pallas_call(
        functools.partial(_combine_body, batch, n),
        in_specs=[
            pl.BlockSpec((batch, 16), lambda: (0, 0)),
            pl.BlockSpec((batch, 128), lambda: (0, 0)),
        ],
        out_specs=pl.BlockSpec((8, 128), lambda: (0, 0)),
        out_shape=jax.ShapeDtypeStruct((8, 128), f32),
    )(tbrem, stats)
    return out[0, 0]


# speedup vs baseline: 30.5120x; 1.1440x over previous
import functools

import jax
import jax.numpy as jnp
from jax import lax
from jax.experimental import pallas as pl
from jax.experimental.pallas import tpu as pltpu
from jax.experimental.pallas import tpu_sc as plsc

K_BT = 3
M_REC = 3
ROWS_A = 8
MXU_M = 5000


def _mxu_body(n_cities, pt_ref, w1t_ref, b1t_ref, wvt_ref, p_ref, o_ref,
              stats_ref, idx_ref):
    w1t = w1t_ref[...]
    b1t = b1t_ref[...]
    wvt = wvt_ref[...]
    for r in range(ROWS_A):
        xt = pt_ref[r]
        ht = jax.lax.dot_general(w1t, xt, (((1,), (0,)), ((), ())),
                                 preferred_element_type=jnp.float32)
        ht = jnp.maximum(ht + b1t, 0.0)
        st = jax.lax.dot_general(wvt, ht, (((1,), (0,)), ((), ())),
                                 preferred_element_type=jnp.float32)
        o_ref[r:r + 1, :] = st

    scores = o_ref[...]
    m = jnp.max(scores, axis=1, keepdims=True)
    se = jnp.sum(jnp.exp(scores - m), axis=1, keepdims=True)
    lse = m + jnp.log(se)
    rowsum = jnp.sum(scores, axis=1, keepdims=True)

    iota = lax.broadcasted_iota(jnp.int32, scores.shape, 1)
    s = scores
    tv, ti = [], []
    for _ in range(K_BT):
        mv = jnp.max(s, axis=1, keepdims=True)
        mi = jnp.min(jnp.where(s == mv, iota, n_cities), axis=1,
                     keepdims=True)
        tv.append(mv)
        ti.append(mi)
        s = jnp.where(iota == mi, -jnp.inf, s)

    col = lax.broadcasted_iota(jnp.int32, stats_ref.shape, 1)
    stats = jnp.zeros(stats_ref.shape, jnp.float32)
    for c, val in ((0, lse), (1, rowsum), (2, tv[0]), (3, tv[1]), (4, tv[2]),
                   (5, jnp.full_like(lse, p_ref[1])),
                   (6, jnp.full_like(lse, p_ref[0]))):
        stats = jnp.where(col == c, val, stats)
    stats_ref[...] = stats
    idx = jnp.zeros(idx_ref.shape, jnp.int32)
    for c, val in ((0, ti[0]), (1, ti[1]), (2, ti[2])):
        idx = jnp.where(col == c, val, idx)
    idx_ref[...] = idx


def _sqrt16(x):
    i = plsc.bitcast(x, jnp.int32)
    y = plsc.bitcast(jnp.int32(0x5F3759DF) - (i >> 1), jnp.float32)
    for _ in range(3):
        y = y * (1.5 - 0.5 * x * y * y)
    return x * y


def _sc_body(n_cities, pt_hbm, ini_hbm, sc_hbm, st_hbm, bt_hbm,
             out_hbm, ini_v, px_v, py_v, tcx_v, tcy_v, sc_v, st_v, bt_v,
             out_v):
    n = n_cities
    wid = lax.axis_index("c") * 16 + lax.axis_index("s")
    iota16 = lax.iota(jnp.int32, 16)
    zeros16 = jnp.zeros((16,), jnp.float32)

    for r in range(2):
        b = wid + 32 * r
        pltpu.sync_copy(ini_hbm.at[b], ini_v)
        pltpu.sync_copy(pt_hbm.at[b, 0], px_v)
        pltpu.sync_copy(pt_hbm.at[b, 1], py_v)
        pltpu.sync_copy(sc_hbm.at[b], sc_v)
        pltpu.sync_copy(st_hbm.at[b], st_v)
        pltpu.sync_copy(bt_hbm.at[b], bt_v)

        def chunk(k, acc):
            off = k * 16
            ia = ini_v[pl.ds(off, 16)]
            ip1 = iota16 + (off + 1)
            ip1 = jnp.where(ip1 == n, 0, ip1)
            ib = plsc.load_gather(ini_v, [ip1])
            xa = plsc.load_gather(px_v, [ia])
            ya = plsc.load_gather(py_v, [ia])
            xb = plsc.load_gather(px_v, [ib])
            yb = plsc.load_gather(py_v, [ib])
            tcx_v[pl.ds(off, 16)] = xa
            tcy_v[pl.ds(off, 16)] = ya
            dx = xa - xb
            dy = ya - yb
            return acc + _sqrt16(dx * dx + dy * dy + 1e-10)

        rem16 = lax.fori_loop(0, n // 16, chunk, zeros16, unroll=2)
        remb = jnp.sum(rem16)

        st16 = st_v[pl.ds(0, 16)]
        bt16 = bt_v[pl.ds(0, 16)]
        lse = st16[0]
        logz = st16[5]
        outvec = zeros16
        for t in range(K_BT):
            lo = bt16[t]
            log_pb = st16[2 + t] - lse
            lom1 = jnp.where(lo == 0, n - 1, lo - 1)
            for j in range(M_REC):
                seg = 100 * (j + 1)
                hi = lo + seg
                w_end = jnp.minimum(hi, n)
                kmax = w_end - lo
                wmod = jnp.where(w_end == n, 0, w_end)

                def cchunk(c, carry, hi=hi, kmax=kmax, wmod=wmod, lo=lo,
                           lom1=lom1):
                    dacc, wacc = carry
                    kv = iota16 + c * 16
                    m_edge = kv <= kmax
                    m_win = kv < kmax
                    rev = jnp.clip(hi - 1 - kv, 0, n - 1)
                    tca = jnp.where(kv == 0, lom1, jnp.clip(hi - kv, 0, n - 1))
                    tcb = jnp.where(kv == kmax, wmod, rev)
                    olda = jnp.where(kv == 0, lom1,
                                     jnp.clip(lo - 1 + kv, 0, n - 1))
                    oldb = jnp.where(kv == kmax, wmod,
                                     jnp.clip(lo + kv, 0, n - 1))
                    nax = plsc.load_gather(tcx_v, [tca])
                    nay = plsc.load_gather(tcy_v, [tca])
                    nbx = plsc.load_gather(tcx_v, [tcb])
                    nby = plsc.load_gather(tcy_v, [tcb])
                    oax = plsc.load_gather(tcx_v, [olda])
                    oay = plsc.load_gather(tcy_v, [olda])
                    obx = plsc.load_gather(tcx_v, [oldb])
                    oby = plsc.load_gather(tcy_v, [oldb])
                    ndx = nax - nbx
                    ndy = nay - nby
                    odx = oax - obx
                    ody = oay - oby
                    nd = _sqrt16(ndx * ndx + ndy * ndy + 1e-10)
                    od = _sqrt16(odx * odx + ody * ody + 1e-10)
                    sv = plsc.load_gather(sc_v, [rev])
                    dacc = dacc + jnp.where(m_edge, nd - od, 0.0)
                    wacc = wacc + jnp.where(m_win, sv, 0.0)
                    return dacc, wacc

                d16, w16 = lax.fori_loop(0, 19, cchunk, (zeros16, zeros16))
                delta = jnp.sum(d16)
                win_logp = jnp.sum(w16) - kmax.astype(jnp.float32) * lse
                tb = logz + win_logp + remb + delta - log_pb
                outvec = jnp.where(iota16 == t * M_REC + j, tb * tb, outvec)
        outvec = jnp.where(iota16 == 9, remb, outvec)
        out_v[pl.ds(0, 16)] = outvec
        pltpu.sync_copy(out_v, out_hbm.at[b])


def _combine_body(batch, n_cities, tb_ref, st_ref, out_ref):
    tb = tb_ref[...]
    st = st_ref[...]
    col = lax.broadcasted_iota(jnp.int32, tb.shape, 1)
    loss_tb = jnp.sum(jnp.where(col < K_BT * M_REC, tb, 0.0)) / (
        batch * K_BT * M_REC)
    pred = st[:, 1:2] / n_cities + st[:, 6:7]
    rem = tb[:, 9:10]
    v_loss = jnp.sum((pred - rem) * (pred - rem)) / batch
    out_ref[...] = jnp.full(out_ref.shape, loss_tb + 0.1 * v_loss,
                            jnp.float32)


def kernel(problems, initial, W1, b1, Wv, bv, logZ):
    batch, n, _ = problems.shape
    f32 = jnp.float32
    pt = problems.transpose(0, 2, 1)
    pack = jnp.concatenate([bv, logZ, jnp.zeros((6,), f32)])

    scores, stats, bt = pl.pallas_call(
        functools.partial(_mxu_body, n),
        grid=(batch // ROWS_A,),
        in_specs=[
            pl.BlockSpec((ROWS_A, 2, n), lambda i: (i, 0, 0)),
            pl.BlockSpec((32, 2), lambda i: (0, 0)),
            pl.BlockSpec((32, 1), lambda i: (0, 0)),
            pl.BlockSpec((1, 32), lambda i: (0, 0)),
            pl.BlockSpec(memory_space=pltpu.SMEM),
        ],
        out_specs=[
            pl.BlockSpec((ROWS_A, n), lambda i: (i, 0)),
            pl.BlockSpec((ROWS_A, 128), lambda i: (i, 0)),
            pl.BlockSpec((ROWS_A, 128), lambda i: (i, 0)),
        ],
        out_shape=[
            jax.ShapeDtypeStruct((batch, n), f32),
            jax.ShapeDtypeStruct((batch, 128), f32),
            jax.ShapeDtypeStruct((batch, 128), jnp.int32),
        ],
    )(pt, W1.T, b1.reshape(32, 1), Wv.T, pack)

    mesh = plsc.VectorSubcoreMesh(core_axis_name="c", subcore_axis_name="s")
    tbrem = pl.kernel(
        functools.partial(_sc_body, n),
        out_type=jax.ShapeDtypeStruct((batch, 16), f32),
        mesh=mesh,
        compiler_params=pltpu.CompilerParams(needs_layout_passes=False),
        scratch_types=[
            pltpu.VMEM((n,), jnp.int32),
            pltpu.VMEM((n,), f32),
            pltpu.VMEM((n,), f32),
            pltpu.VMEM((n,), f32),
            pltpu.VMEM((n,), f32),
            pltpu.VMEM((n,), f32),
            pltpu.VMEM((128,), f32),
            pltpu.VMEM((128,), jnp.int32),
            pltpu.VMEM((16,), f32),
        ],
    )(pt, initial, scores, stats, bt)

    out = pl.pallas_call(
        functools.partial(_combine_body, batch, n),
        in_specs=[
            pl.BlockSpec((batch, 16), lambda: (0, 0)),
            pl.BlockSpec((batch, 128), lambda: (0, 0)),
        ],
        out_specs=pl.BlockSpec((8, 128), lambda: (0, 0)),
        out_shape=jax.ShapeDtypeStruct((8, 128), f32),
    )(tbrem, stats)
    return out[0, 0]
